# baseline (speedup 1.0000x reference)
import jax
import jax.numpy as jnp
from jax import lax
from jax.experimental import pallas as pl
from jax.experimental.pallas import tpu as pltpu
from jax.experimental.pallas import tpu_sc as plsc

VOCAB = 49408
N_EMBED = 1024
N_TOKENS = 77
BATCH = 1024
N = BATCH * N_TOKENS
NC = 2
NS = 16
NW = NC * NS
ROWS_PER_W = N // NW
CHUNK = 8
NBUF = 4
LOOKAHEAD = 2
CHUNKS = ROWS_PER_W // CHUNK
OUTER = CHUNKS // NBUF
LANES = 16


def _emb_body(tok_hbm, table_hbm, pos_hbm, out_hbm,
              idx_v, pos_v, buf_v,
              sg0, sg1, sg2, sg3, ss0, ss1, ss2, ss3):
    sem_g = (sg0, sg1, sg2, sg3)
    sem_s = (ss0, ss1, ss2, ss3)
    wid = lax.axis_index("s") * NC + lax.axis_index("c")
    base = wid * ROWS_PER_W
    pltpu.sync_copy(tok_hbm.at[pl.ds(base, ROWS_PER_W)], idx_v)
    pltpu.sync_copy(pos_hbm, pos_v)

    def start_gather(chunk, slot):
        idx = idx_v.at[pl.ds(chunk * CHUNK, CHUNK)]
        pltpu.async_copy(table_hbm.at[idx], buf_v.at[slot], sem_g[slot])

    def wait_gather(slot):
        pltpu.make_async_copy(
            table_hbm.at[idx_v.at[pl.ds(0, CHUNK)]], buf_v.at[slot],
            sem_g[slot]).wait()

    def start_store(chunk, slot):
        pltpu.async_copy(buf_v.at[slot],
                         out_hbm.at[pl.ds(base + chunk * CHUNK, CHUNK)],
                         sem_s[slot])

    def wait_store(slot):
        pltpu.make_async_copy(buf_v.at[slot],
                              out_hbm.at[pl.ds(base, CHUNK)],
                              sem_s[slot]).wait()

    start_gather(0, 0)
    start_gather(1, 1)

    def outer_body(o, carry):
        for b in range(NBUF):
            cc = o * NBUF + b
            wait_gather(b)
            b2 = (b + LOOKAHEAD) % NBUF
            if b < LOOKAHEAD:
                @pl.when(o > 0)
                def _():
                    wait_store(b2)
                start_gather(cc + LOOKAHEAD, b2)
            else:
                wait_store(b2)
                @pl.when(o < OUTER - 1)
                def _():
                    start_gather(cc + LOOKAHEAD, b2)
            def row_body(j, _):
                t = lax.rem(cc * CHUNK + j, N_TOKENS)
                for s in range(N_EMBED // LANES):
                    sl = pl.ds(s * LANES, LANES)
                    plsc.addupdate(buf_v.at[b, j, sl], pos_v[t, sl])
                return 0
            lax.fori_loop(0, CHUNK, row_body, 0)
            start_store(cc, b)
        return carry

    lax.fori_loop(0, OUTER, outer_body, 0)
    wait_store(2)
    wait_store(3)


def kernel(tokens, token_embedding, position_embedding):
    tok_flat = tokens.reshape(-1).astype(jnp.int32)
    mesh = plsc.VectorSubcoreMesh(core_axis_name="c", subcore_axis_name="s")
    out = pl.kernel(
        _emb_body,
        mesh=mesh,
        out_type=jax.ShapeDtypeStruct((N, N_EMBED), jnp.float32),
        scratch_types=[
            pltpu.VMEM((ROWS_PER_W,), jnp.int32),
            pltpu.VMEM((N_TOKENS, N_EMBED), jnp.float32),
            pltpu.VMEM((NBUF, CHUNK, N_EMBED), jnp.float32),
        ] + [pltpu.SemaphoreType.DMA] * 8,
    )(tok_flat, token_embedding, position_embedding)
    return out.reshape(BATCH, N_TOKENS, N_EMBED)


# speedup vs baseline: 1.6296x; 1.6296x over previous
import jax
import jax.numpy as jnp
from jax import lax
from jax.experimental import pallas as pl
from jax.experimental.pallas import tpu as pltpu
from jax.experimental.pallas import tpu_sc as plsc

VOCAB = 49408
N_EMBED = 1024
N_TOKENS = 77
BATCH = 1024
NC = 2
NS = 16
NW = NC * NS
B_PER_W = BATCH // NW
CHUNK = 8
FULL_CHUNKS = N_TOKENS // CHUNK
TAIL = N_TOKENS - FULL_CHUNKS * CHUNK
T0 = FULL_CHUNKS * CHUNK
NBUF = 4
LOOKAHEAD = 2
VISITS = B_PER_W * FULL_CHUNKS
OUTER = VISITS // NBUF
LANES = 16
T_PAD = 80


def _emb_body(tok_hbm, table_hbm, pos_hbm, out_hbm,
              idx_v, posbuf_v, tailpos_v, buf_v, tail0_v, tail1_v,
              sg0, sg1, sg2, sg3, ss0, ss1, ss2, ss3):
    tails = (tail0_v, tail1_v)
    sem_g = (sg0, sg1, sg2, sg3)
    sem_s = (ss0, ss1, ss2, ss3)
    wid = lax.axis_index("s") * NC + lax.axis_index("c")
    bb0 = wid * B_PER_W
    pltpu.sync_copy(tok_hbm.at[pl.ds(bb0 * T_PAD, B_PER_W * T_PAD)], idx_v)

    def visit_coords(v):
        c = v // B_PER_W
        k = v - c * B_PER_W
        return k, c * CHUNK

    def start_gather(v, slot):
        k, t0 = visit_coords(v)
        idx = idx_v.at[pl.ds(k * T_PAD + t0, CHUNK)]
        pltpu.async_copy(table_hbm.at[idx], buf_v.at[slot], sem_g[slot])

    def wait_gather(slot):
        pltpu.make_async_copy(
            table_hbm.at[idx_v.at[pl.ds(0, CHUNK)]],
            buf_v.at[slot], sem_g[slot]).wait()

    def start_store(v, slot):
        k, t0 = visit_coords(v)
        pltpu.async_copy(buf_v.at[slot],
                         out_hbm.at[bb0 + k, pl.ds(t0, CHUNK)],
                         sem_s[slot])

    def wait_store(slot):
        pltpu.make_async_copy(buf_v.at[slot],
                              out_hbm.at[bb0, pl.ds(0, CHUNK)],
                              sem_s[slot]).wait()

    def add_pos(slot):
        def row_body(j, _):
            for s in range(N_EMBED // LANES):
                sl = pl.ds(s * LANES, LANES)
                plsc.addupdate(buf_v.at[slot, j, sl], posbuf_v[j, sl])
            return 0
        lax.fori_loop(0, CHUNK, row_body, 0)

    pltpu.sync_copy(pos_hbm.at[pl.ds(0, CHUNK)], posbuf_v)
    start_gather(0, 0)
    start_gather(1, 1)

    def outer_body(o, carry):
        for b in range(NBUF):
            cc = o * NBUF + b
            wait_gather(b)
            b2 = (b + LOOKAHEAD) % NBUF
            if b < LOOKAHEAD:
                @pl.when(o > 0)
                def _():
                    wait_store(b2)
                start_gather(cc + LOOKAHEAD, b2)
            else:
                wait_store(b2)
                @pl.when(o < OUTER - 1)
                def _():
                    start_gather(cc + LOOKAHEAD, b2)
            k, t0 = visit_coords(cc)

            @pl.when(k == 0)
            def _():
                pltpu.sync_copy(pos_hbm.at[pl.ds(t0, CHUNK)], posbuf_v)

            add_pos(b)
            start_store(cc, b)
        return carry

    lax.fori_loop(0, OUTER, outer_body, 0)
    wait_store(2)
    wait_store(3)

    def tail_gather(k, slot):
        idx = idx_v.at[pl.ds(k * T_PAD + T0, CHUNK)]
        pltpu.async_copy(table_hbm.at[idx], tails[slot], sem_g[slot])

    def tail_wait_gather(slot):
        pltpu.make_async_copy(table_hbm.at[idx_v.at[pl.ds(T0, CHUNK)]],
                              tails[slot], sem_g[slot]).wait()

    def tail_wait_store(slot):
        pltpu.make_async_copy(tails[slot].at[pl.ds(0, TAIL)],
                              out_hbm.at[bb0, pl.ds(T0, TAIL)],
                              sem_s[slot]).wait()

    def tail_add(slot):
        def row_body(j, _):
            for s in range(N_EMBED // LANES):
                sl = pl.ds(s * LANES, LANES)
                plsc.addupdate(tails[slot].at[j, sl], tailpos_v[j, sl])
            return 0
        lax.fori_loop(0, TAIL, row_body, 0)

    pltpu.sync_copy(pos_hbm.at[pl.ds(T0, TAIL)], tailpos_v)
    tail_gather(0, 0)

    def tail_pair(p, carry):
        for b in range(2):
            k = p * 2 + b
            tail_wait_gather(b)

            @pl.when(k + 1 < B_PER_W)
            def _():
                @pl.when(k >= 1)
                def _():
                    tail_wait_store(1 - b)
                tail_gather(k + 1, 1 - b)

            tail_add(b)
            pltpu.async_copy(tails[b].at[pl.ds(0, TAIL)],
                             out_hbm.at[bb0 + k, pl.ds(T0, TAIL)],
                             sem_s[b])
        return carry

    lax.fori_loop(0, B_PER_W // 2, tail_pair, 0)
    tail_wait_store(0)
    tail_wait_store(1)


def kernel(tokens, token_embedding, position_embedding):
    tok_pad = jnp.pad(tokens.astype(jnp.int32),
                      ((0, 0), (0, T_PAD - N_TOKENS))).reshape(-1)
    mesh = plsc.VectorSubcoreMesh(core_axis_name="c", subcore_axis_name="s")
    out = pl.kernel(
        _emb_body,
        mesh=mesh,
        out_type=jax.ShapeDtypeStruct((BATCH, N_TOKENS, N_EMBED), jnp.float32),
        scratch_types=[
            pltpu.VMEM((B_PER_W * T_PAD,), jnp.int32),
            pltpu.VMEM((CHUNK, N_EMBED), jnp.float32),
            pltpu.VMEM((TAIL, N_EMBED), jnp.float32),
            pltpu.VMEM((NBUF, CHUNK, N_EMBED), jnp.float32),
            pltpu.VMEM((CHUNK, N_EMBED), jnp.float32),
            pltpu.VMEM((CHUNK, N_EMBED), jnp.float32),
        ] + [pltpu.SemaphoreType.DMA] * 8,
    )(tok_pad, token_embedding, position_embedding)
    return out


# speedup vs baseline: 1.6316x; 1.0012x over previous
import jax
import jax.numpy as jnp
from jax import lax
from jax.experimental import pallas as pl
from jax.experimental.pallas import tpu as pltpu
from jax.experimental.pallas import tpu_sc as plsc

VOCAB = 49408
N_EMBED = 1024
N_TOKENS = 77
BATCH = 1024
NC = 2
NS = 16
NW = NC * NS
B_PER_W = BATCH // NW
CHUNK = 8
FULL_CHUNKS = N_TOKENS // CHUNK
TAIL = N_TOKENS - FULL_CHUNKS * CHUNK
T0 = FULL_CHUNKS * CHUNK
NBUF = 8
LOOKAHEAD = 4
VISITS = B_PER_W * FULL_CHUNKS
OUTER = VISITS // NBUF
LANES = 16
T_PAD = 80


def _emb_body(tok_hbm, table_hbm, pos_hbm, out_hbm,
              idx_v, posbuf_v, tailpos_v, buf_v, tail0_v, tail1_v,
              sg0, sg1, sg2, sg3, sg4, sg5, sg6, sg7,
              ss0, ss1, ss2, ss3, ss4, ss5, ss6, ss7):
    tails = (tail0_v, tail1_v)
    sem_g = (sg0, sg1, sg2, sg3, sg4, sg5, sg6, sg7)
    sem_s = (ss0, ss1, ss2, ss3, ss4, ss5, ss6, ss7)
    wid = lax.axis_index("s") * NC + lax.axis_index("c")
    bb0 = wid * B_PER_W
    pltpu.sync_copy(tok_hbm.at[pl.ds(bb0 * T_PAD, B_PER_W * T_PAD)], idx_v)

    def visit_coords(v):
        c = v // B_PER_W
        k = v - c * B_PER_W
        return k, c * CHUNK

    def start_gather(v, slot):
        k, t0 = visit_coords(v)
        idx = idx_v.at[pl.ds(k * T_PAD + t0, CHUNK)]
        pltpu.async_copy(table_hbm.at[idx], buf_v.at[slot], sem_g[slot])

    def wait_gather(slot):
        pltpu.make_async_copy(
            table_hbm.at[idx_v.at[pl.ds(0, CHUNK)]],
            buf_v.at[slot], sem_g[slot]).wait()

    def start_store(v, slot):
        k, t0 = visit_coords(v)
        pltpu.async_copy(buf_v.at[slot],
                         out_hbm.at[bb0 + k, pl.ds(t0, CHUNK)],
                         sem_s[slot])

    def wait_store(slot):
        pltpu.make_async_copy(buf_v.at[slot],
                              out_hbm.at[bb0, pl.ds(0, CHUNK)],
                              sem_s[slot]).wait()

    def add_pos(slot):
        def row_body(j, _):
            for s in range(N_EMBED // LANES):
                sl = pl.ds(s * LANES, LANES)
                plsc.addupdate(buf_v.at[slot, j, sl], posbuf_v[j, sl])
            return 0
        lax.fori_loop(0, CHUNK, row_body, 0)

    pltpu.sync_copy(pos_hbm.at[pl.ds(0, CHUNK)], posbuf_v)
    for v in range(LOOKAHEAD):
        start_gather(v, v)

    def outer_body(o, carry):
        for b in range(NBUF):
            cc = o * NBUF + b
            wait_gather(b)
            b2 = (b + LOOKAHEAD) % NBUF
            if b < LOOKAHEAD:
                @pl.when(o > 0)
                def _():
                    wait_store(b2)
                start_gather(cc + LOOKAHEAD, b2)
            else:
                wait_store(b2)
                @pl.when(o < OUTER - 1)
                def _():
                    start_gather(cc + LOOKAHEAD, b2)
            k, t0 = visit_coords(cc)

            @pl.when(k == 0)
            def _():
                pltpu.sync_copy(pos_hbm.at[pl.ds(t0, CHUNK)], posbuf_v)

            add_pos(b)
            start_store(cc, b)
        return carry

    lax.fori_loop(0, OUTER, outer_body, 0)
    for slot in range(NBUF - LOOKAHEAD, NBUF):
        wait_store(slot)

    def tail_gather(k, slot):
        idx = idx_v.at[pl.ds(k * T_PAD + T0, CHUNK)]
        pltpu.async_copy(table_hbm.at[idx], tails[slot], sem_g[slot])

    def tail_wait_gather(slot):
        pltpu.make_async_copy(table_hbm.at[idx_v.at[pl.ds(T0, CHUNK)]],
                              tails[slot], sem_g[slot]).wait()

    def tail_wait_store(slot):
        pltpu.make_async_copy(tails[slot].at[pl.ds(0, TAIL)],
                              out_hbm.at[bb0, pl.ds(T0, TAIL)],
                              sem_s[slot]).wait()

    def tail_add(slot):
        def row_body(j, _):
            for s in range(N_EMBED // LANES):
                sl = pl.ds(s * LANES, LANES)
                plsc.addupdate(tails[slot].at[j, sl], tailpos_v[j, sl])
            return 0
        lax.fori_loop(0, TAIL, row_body, 0)

    pltpu.sync_copy(pos_hbm.at[pl.ds(T0, TAIL)], tailpos_v)
    tail_gather(0, 0)

    def tail_pair(p, carry):
        for b in range(2):
            k = p * 2 + b
            tail_wait_gather(b)

            @pl.when(k + 1 < B_PER_W)
            def _():
                @pl.when(k >= 1)
                def _():
                    tail_wait_store(1 - b)
                tail_gather(k + 1, 1 - b)

            tail_add(b)
            pltpu.async_copy(tails[b].at[pl.ds(0, TAIL)],
                             out_hbm.at[bb0 + k, pl.ds(T0, TAIL)],
                             sem_s[b])
        return carry

    lax.fori_loop(0, B_PER_W // 2, tail_pair, 0)
    tail_wait_store(0)
    tail_wait_store(1)


def kernel(tokens, token_embedding, position_embedding):
    tok_pad = jnp.pad(tokens.astype(jnp.int32),
                      ((0, 0), (0, T_PAD - N_TOKENS))).reshape(-1)
    mesh = plsc.VectorSubcoreMesh(core_axis_name="c", subcore_axis_name="s")
    out = pl.kernel(
        _emb_body,
        mesh=mesh,
        out_type=jax.ShapeDtypeStruct((BATCH, N_TOKENS, N_EMBED), jnp.float32),
        scratch_types=[
            pltpu.VMEM((B_PER_W * T_PAD,), jnp.int32),
            pltpu.VMEM((CHUNK, N_EMBED), jnp.float32),
            pltpu.VMEM((TAIL, N_EMBED), jnp.float32),
            pltpu.VMEM((NBUF, CHUNK, N_EMBED), jnp.float32),
            pltpu.VMEM((CHUNK, N_EMBED), jnp.float32),
            pltpu.VMEM((CHUNK, N_EMBED), jnp.float32),
        ] + [pltpu.SemaphoreType.DMA] * 16,
    )(tok_pad, token_embedding, position_embedding)
    return out
